# Initial kernel scaffold; baseline (speedup 1.0000x reference)
import jax
import jax.numpy as jnp
from jax.experimental import pallas as pl


def kernel(x, target):
    raise NotImplementedError("write your pallas kernel here")


# speedup vs baseline: 1.6922x; 1.6922x over previous
import functools
import math

import jax
import jax.numpy as jnp
from jax import lax
from jax.experimental import pallas as pl
from jax.experimental.pallas import tpu as pltpu
from jax.experimental.pallas import tpu_sc as plsc

N = 1024
V = 100000
PAD = 0
EPS = 0.1 / (V - 1)
CONF = 0.9
C1 = (V - 2) * EPS * math.log(EPS) + CONF * math.log(CONF)

_L = 16
_NW = 32
_RPW = N // _NW
_BLK = V // _L

_BR = 256
_BC = 4096
_NR = N // _BR
_NC = (V + _BC - 1) // _BC


def _sc_gather(xe, tgt):
    mesh = plsc.VectorSubcoreMesh(core_axis_name="c", subcore_axis_name="s")

    @functools.partial(
        pl.kernel,
        mesh=mesh,
        out_type=(
            jax.ShapeDtypeStruct((N, 1), jnp.float32),
            jax.ShapeDtypeStruct((N, 1), jnp.float32),
        ),
        scratch_types=[
            pltpu.VMEM((_RPW,), jnp.int32),
            pltpu.VMEM((_RPW,), jnp.int32),
            pltpu.VMEM((_RPW,), jnp.int32),
            pltpu.VMEM((_RPW, 1), jnp.float32),
            pltpu.VMEM((_RPW, 1), jnp.float32),
            pltpu.SemaphoreType.DMA,
            pltpu.SemaphoreType.DMA,
        ],
    )
    def k(x_hbm, t_hbm, g_hbm, z_hbm, tgt_v, gidx_v, zidx_v,
          grow_v, zrow_v, sem_g, sem_z):
        wid = lax.axis_index("s") * 2 + lax.axis_index("c")
        base = wid * _RPW
        pltpu.sync_copy(t_hbm.at[pl.ds(base, _RPW)], tgt_v)
        for c in range(_RPW // _L):
            lane = lax.iota(jnp.int32, _L)
            rid = base + c * _L + lane
            t = tgt_v[pl.ds(c * _L, _L)]
            gidx_v[pl.ds(c * _L, _L)] = rid * V + t
            zidx_v[pl.ds(c * _L, _L)] = rid * V
        cp_g = pltpu.async_copy(x_hbm.at[gidx_v], grow_v, sem_g)
        cp_z = pltpu.async_copy(x_hbm.at[zidx_v], zrow_v, sem_z)
        cp_g.wait()
        cp_z.wait()
        pltpu.sync_copy(grow_v, g_hbm.at[pl.ds(base, _RPW), :])
        pltpu.sync_copy(zrow_v, z_hbm.at[pl.ds(base, _RPW), :])

    return k(xe, tgt)


def _tc_body(x_ref, t_ref, g_ref, z_ref, out_ref):
    i = pl.program_id(0)
    j = pl.program_id(1)

    @pl.when((i == 0) & (j == 0))
    def _init():
        out_ref[...] = jnp.zeros((1, 1), jnp.float32)

    nonpad = (t_ref[...] != PAD).astype(jnp.float32)

    @pl.when(j == 0)
    def _row_consts():
        per_row = C1 + EPS * z_ref[...] - (CONF - EPS) * g_ref[...]
        out_ref[...] += jnp.sum(per_row * nonpad).reshape(1, 1)

    colmask = (j * _BC + lax.broadcasted_iota(jnp.int32, (1, _BC), 1)) < V
    xb = jnp.where(colmask, x_ref[...], 0.0)
    rowsum = jnp.sum(xb, axis=1, keepdims=True)
    out_ref[...] += (-EPS * jnp.sum(rowsum * nonpad)).reshape(1, 1)


def _tc_body_full(x_ref, t_ref, out_ref):
    i = pl.program_id(0)
    j = pl.program_id(1)

    @pl.when((i == 0) & (j == 0))
    def _init():
        out_ref[...] = jnp.zeros((1, 1), jnp.float32)

    tgt = t_ref[...]
    nonpad = (tgt != PAD).astype(jnp.float32)
    col = j * _BC + lax.broadcasted_iota(jnp.int32, (1, _BC), 1)
    xb = jnp.where(col < V, x_ref[...], 0.0)
    rowsum = jnp.sum(xb, axis=1, keepdims=True)
    g_here = jnp.sum(jnp.where(col == tgt, xb, 0.0), axis=1, keepdims=True)
    acc = jnp.sum((-EPS * rowsum - (CONF - EPS) * g_here) * nonpad)

    @pl.when(j == 0)
    def _row_consts():
        per_row = C1 + EPS * x_ref[:, 0:1]
        out_ref[...] += jnp.sum(per_row * nonpad).reshape(1, 1)

    out_ref[...] += acc.reshape(1, 1)


def kernel(x, target):
    tgt = target.astype(jnp.int32)
    loss = pl.pallas_call(
        _tc_body_full,
        grid=(_NR, _NC),
        in_specs=[
            pl.BlockSpec((_BR, _BC), lambda i, j: (i, j)),
            pl.BlockSpec((_BR, 1), lambda i, j: (i, 0)),
        ],
        out_specs=pl.BlockSpec((1, 1), lambda i, j: (0, 0)),
        out_shape=jax.ShapeDtypeStruct((1, 1), jnp.float32),
    )(x, tgt.reshape(N, 1))
    return jnp.reshape(loss, ())


# speedup vs baseline: 1.8438x; 1.0896x over previous
import math

import jax
import jax.numpy as jnp
from jax import lax
from jax.experimental import pallas as pl

N = 1024
V = 100000
PAD = 0
EPS = 0.1 / (V - 1)
CONF = 0.9
C1 = (V - 2) * EPS * math.log(EPS) + CONF * math.log(CONF)

_BC = 4096
_NC = (V + _BC - 1) // _BC


def _tc_body_full(x_ref, t_ref, out_ref):
    j = pl.program_id(0)

    @pl.when(j == 0)
    def _init():
        out_ref[...] = jnp.zeros((1, 1), jnp.float32)

    tgt = t_ref[...]
    nonpad = (tgt != PAD).astype(jnp.float32)
    col = j * _BC + lax.broadcasted_iota(jnp.int32, (1, _BC), 1)
    xb = jnp.where(col < V, x_ref[...], 0.0)
    rowsum = jnp.sum(xb, axis=1, keepdims=True)
    g_here = jnp.sum(jnp.where(col == tgt, xb, 0.0), axis=1, keepdims=True)
    acc = jnp.sum((-EPS * rowsum - (CONF - EPS) * g_here) * nonpad)

    @pl.when(j == 0)
    def _row_consts():
        per_row = C1 + EPS * x_ref[:, 0:1]
        out_ref[...] += jnp.sum(per_row * nonpad).reshape(1, 1)

    out_ref[...] += acc.reshape(1, 1)


def kernel(x, target):
    tgt = target.astype(jnp.int32)
    loss = pl.pallas_call(
        _tc_body_full,
        grid=(_NC,),
        in_specs=[
            pl.BlockSpec((N, _BC), lambda j: (0, j)),
            pl.BlockSpec((N, 1), lambda j: (0, 0)),
        ],
        out_specs=pl.BlockSpec((1, 1), lambda j: (0, 0)),
        out_shape=jax.ShapeDtypeStruct((1, 1), jnp.float32),
    )(x, tgt.reshape(N, 1))
    return jnp.reshape(loss, ())
